# baseline (speedup 1.0000x reference)
import functools

import jax
import jax.numpy as jnp
from jax import lax
from jax.experimental import pallas as pl
from jax.experimental.pallas import tpu as pltpu
from jax.experimental.pallas import tpu_sc as plsc

N = 10000
NP_ = 10240
E = 320000
NC = 2
NS = 16
CH = 128
RCH = 8
ROWS = 2560
EP = ROWS * CH
NPT = NP_ // NS
D = 128

_EPS = 1e-5


def _sc_mesh():
    return plsc.VectorSubcoreMesh(core_axis_name="c", subcore_axis_name="s")


@functools.partial(
    pl.kernel,
    out_type=jax.ShapeDtypeStruct((NC, NP_, D), jnp.float32),
    mesh=_sc_mesh(),
    scratch_types=[
        pltpu.VMEM((RCH, CH), jnp.int32),
        pltpu.VMEM((CH, D), jnp.float32),
        pltpu.VMEM_SHARED((NP_, D), jnp.float32),
    ],
    name="gcn_degrees",
)
def _deg_kernel(edge_hbm, ones_hbm, zeros_hbm, out_hbm, idx_v, ones_v, tab):
    c = lax.axis_index("c")
    s = lax.axis_index("s")
    rpt = ROWS // NS
    pltpu.sync_copy(zeros_hbm, tab.at[pl.ds(s * NPT, NPT)])
    pltpu.sync_copy(ones_hbm, ones_v)
    plsc.subcore_barrier()

    def body(k, carry):
        r0 = s * rpt + k * RCH
        pltpu.sync_copy(edge_hbm.at[c].at[pl.ds(r0, RCH)], idx_v)
        for j in range(RCH):
            pltpu.sync_copy(ones_v, tab.at[idx_v.at[j]], add=True)
        return carry

    lax.fori_loop(0, rpt // RCH, body, 0)
    plsc.subcore_barrier()
    pltpu.sync_copy(tab.at[pl.ds(s * NPT, NPT)],
                    out_hbm.at[c].at[pl.ds(s * NPT, NPT)])


def _make_agg(edge_split):
    @functools.partial(
        pl.kernel,
        out_type=jax.ShapeDtypeStruct((NC, NP_, D), jnp.float32),
        mesh=_sc_mesh(),
        scratch_types=[
            pltpu.VMEM((RCH, CH), jnp.int32),
            pltpu.VMEM((RCH, CH), jnp.int32),
            pltpu.VMEM((2, CH, D), jnp.float32),
            pltpu.VMEM_SHARED((NP_, D), jnp.float32),
            pltpu.SemaphoreType.DMA,
        ],
        name=f"gcn_agg_{'es' if edge_split else 'fs'}",
    )
    def agg(hs_hbm, src_hbm, dst_hbm, zeros_hbm, out_hbm, sidx, didx, rows, acc, sem):
        c = lax.axis_index("c")
        s = lax.axis_index("s")
        if edge_split:
            rpt = ROWS // (NC * NS)
            base = c * (ROWS // NC) + s * rpt
        else:
            rpt = ROWS // NS
            base = s * rpt
        table = hs_hbm.at[c]
        pltpu.sync_copy(zeros_hbm, acc.at[pl.ds(s * NPT, NPT)])
        plsc.subcore_barrier()

        def body(k, carry):
            r0 = base + k * RCH
            pltpu.sync_copy(src_hbm.at[pl.ds(r0, RCH)], sidx)
            pltpu.sync_copy(dst_hbm.at[pl.ds(r0, RCH)], didx)
            for g in range(RCH // 2):
                j0 = g * 2
                cps = [pltpu.async_copy(table.at[sidx.at[j0 + j]], rows.at[j], sem)
                       for j in range(2)]
                for cp in cps:
                    cp.wait()
                for j in range(2):
                    pltpu.sync_copy(rows.at[j], acc.at[didx.at[j0 + j]], add=True)
            return carry

        lax.fori_loop(0, rpt // RCH, body, 0)
        plsc.subcore_barrier()
        pltpu.sync_copy(acc.at[pl.ds(s * NPT, NPT)],
                        out_hbm.at[c].at[pl.ds(s * NPT, NPT)])

    return agg


_agg_es = _make_agg(True)
_agg_fs = _make_agg(False)


def _stage_a(deg_ref, x_ref, hs1_ref, norms_ref):
    ns = lax.rsqrt(jnp.clip(deg_ref[:, 0:1], 1.0, None))
    nd = lax.rsqrt(jnp.clip(deg_ref[:, 1:2], 1.0, None))
    norms_ref[:, 0:1] = ns
    norms_ref[:, 1:2] = nd
    xs = x_ref[...] * ns
    hs1_ref[0, 0:N] = xs
    hs1_ref[1, 0:N] = xs


def _stage_b(agg_ref, norms_ref, w_ref, b_ref, g_ref, be_ref, out_ref):
    ns = norms_ref[:, 0:1]
    nd = norms_ref[:, 1:2]
    a = (agg_ref[0][:N] + agg_ref[1][:N]) * nd
    z = jnp.dot(a, w_ref[...], preferred_element_type=jnp.float32) + b_ref[...]
    m = jnp.mean(z, axis=0, keepdims=True)
    d = z - m
    v = jnp.mean(d * d, axis=0, keepdims=True)
    h = jnp.maximum(d * lax.rsqrt(v + _EPS) * g_ref[...] + be_ref[...], 0.0)
    hs = h * ns
    out_ref[0, 0:N] = hs[:, :D]
    out_ref[1, 0:N] = hs[:, D:]


def _stage_c(agg_ref, norms_ref, w2_ref, b_ref, g_ref, be_ref, w3_ref, out_ref):
    ns = norms_ref[:, 0:1]
    nd = norms_ref[:, 1:2]
    a0 = agg_ref[0][:N] * nd
    a1 = agg_ref[1][:N] * nd
    z = (jnp.dot(a0, w2_ref[0:D, :], preferred_element_type=jnp.float32)
         + jnp.dot(a1, w2_ref[D:2 * D, :], preferred_element_type=jnp.float32)
         + b_ref[...])
    m = jnp.mean(z, axis=0, keepdims=True)
    d = z - m
    v = jnp.mean(d * d, axis=0, keepdims=True)
    h = jnp.maximum(d * lax.rsqrt(v + _EPS) * g_ref[...] + be_ref[...], 0.0)
    t3 = jnp.dot(h, w3_ref[...], preferred_element_type=jnp.float32) * ns
    out_ref[0, 0:N] = t3
    out_ref[1, 0:N] = t3


def _stage_d(agg_ref, norms_ref, b3_ref, out_ref):
    nd = norms_ref[:, 1:2]
    a = agg_ref[0][:N, :40] + agg_ref[1][:N, :40]
    out_ref[...] = a * nd + b3_ref[...]


def kernel(x, edge_index, W1, b1, gamma1, beta1, W2, b2, gamma2, beta2, W3, b3):
    f32 = jnp.float32
    i32 = jnp.int32
    src = edge_index[0].astype(i32)
    dst = edge_index[1].astype(i32)
    pad = jnp.full((EP - E,), N, i32)
    src2d = jnp.concatenate([src, pad]).reshape(ROWS, CH)
    dst2d = jnp.concatenate([dst, pad]).reshape(ROWS, CH)
    edge2d = jnp.stack([src2d, dst2d])
    ones8 = jnp.zeros((CH, D), f32).at[:, 0].set(1.0)
    zrow = jnp.zeros((NPT, D), f32)

    degtab = _deg_kernel(edge2d, ones8, zrow)
    degcols = jnp.stack([degtab[0, :N, 0], degtab[1, :N, 0]], axis=1)

    hs1, norms = pl.pallas_call(
        _stage_a,
        out_shape=[jax.ShapeDtypeStruct((NC, NP_, D), f32),
                   jax.ShapeDtypeStruct((N, 2), f32)],
    )(degcols, x)

    agg1 = _agg_es(hs1, src2d, dst2d, zrow)

    hs2 = pl.pallas_call(
        _stage_b,
        out_shape=jax.ShapeDtypeStruct((NC, NP_, D), f32),
    )(agg1, norms, W1, b1.reshape(1, -1), gamma1.reshape(1, -1),
      beta1.reshape(1, -1))

    agg2 = _agg_fs(hs2, src2d, dst2d, zrow)

    W3p = jnp.zeros((256, D), f32).at[:, :40].set(W3)
    t3 = pl.pallas_call(
        _stage_c,
        out_shape=jax.ShapeDtypeStruct((NC, NP_, D), f32),
    )(agg2, norms, W2, b2.reshape(1, -1), gamma2.reshape(1, -1),
      beta2.reshape(1, -1), W3p)

    agg3 = _agg_es(t3, src2d, dst2d, zrow)

    out = pl.pallas_call(
        _stage_d,
        out_shape=jax.ShapeDtypeStruct((N, 40), f32),
    )(agg3, norms, b3.reshape(1, -1))
    return out


# speedup vs baseline: 1.2401x; 1.2401x over previous
import functools

import jax
import jax.numpy as jnp
from jax import lax
from jax.experimental import pallas as pl
from jax.experimental.pallas import tpu as pltpu
from jax.experimental.pallas import tpu_sc as plsc

N = 10000
NP_ = 10240
E = 320000
NC = 2
NS = 16
CH = 128
RCH = 8
ROWS = 2560
EP = ROWS * CH
NPT = NP_ // NS
D = 128

_EPS = 1e-5


def _sc_mesh():
    return plsc.VectorSubcoreMesh(core_axis_name="c", subcore_axis_name="s")


@functools.partial(
    pl.kernel,
    out_type=jax.ShapeDtypeStruct((NC, NP_, D), jnp.float32),
    mesh=_sc_mesh(),
    scratch_types=[
        pltpu.VMEM((RCH, CH), jnp.int32),
        pltpu.VMEM((CH, D), jnp.float32),
        pltpu.VMEM_SHARED((NP_, D), jnp.float32),
    ],
    name="gcn_degrees",
)
def _deg_kernel(edge_hbm, ones_hbm, zeros_hbm, out_hbm, idx_v, ones_v, tab):
    c = lax.axis_index("c")
    s = lax.axis_index("s")
    rpt = ROWS // NS
    pltpu.sync_copy(zeros_hbm, tab.at[pl.ds(s * NPT, NPT)])
    pltpu.sync_copy(ones_hbm, ones_v)
    plsc.subcore_barrier()

    def body(k, carry):
        r0 = s * rpt + k * RCH
        pltpu.sync_copy(edge_hbm.at[c].at[pl.ds(r0, RCH)], idx_v)
        for j in range(RCH):
            pltpu.sync_copy(ones_v, tab.at[idx_v.at[j]], add=True)
        return carry

    lax.fori_loop(0, rpt // RCH, body, 0)
    plsc.subcore_barrier()
    pltpu.sync_copy(tab.at[pl.ds(s * NPT, NPT)],
                    out_hbm.at[c].at[pl.ds(s * NPT, NPT)])


def _make_agg(edge_split):
    @functools.partial(
        pl.kernel,
        out_type=jax.ShapeDtypeStruct((NC, NP_, D), jnp.float32),
        mesh=_sc_mesh(),
        scratch_types=[
            pltpu.VMEM((RCH, CH), jnp.int32),
            pltpu.VMEM((RCH, CH), jnp.int32),
            pltpu.VMEM((2, CH, D), jnp.float32),
            pltpu.VMEM_SHARED((NP_, D), jnp.float32),
            pltpu.SemaphoreType.DMA,
        ],
        name=f"gcn_agg_{'es' if edge_split else 'fs'}",
    )
    def agg(hs_hbm, src_hbm, dst_hbm, zeros_hbm, out_hbm, sidx, didx, rows, acc, sem):
        c = lax.axis_index("c")
        s = lax.axis_index("s")
        if edge_split:
            rpt = ROWS // (NC * NS)
            base = None
        else:
            rpt = ROWS // NS
            base = s * rpt
        table = hs_hbm.at[c]
        pltpu.sync_copy(zeros_hbm, acc.at[pl.ds(s * NPT, NPT)])
        plsc.subcore_barrier()

        def body(k, carry):
            if edge_split:
                r0 = ((s * (rpt // RCH) + k) * NC + c) * RCH
            else:
                r0 = base + k * RCH
            pltpu.sync_copy(src_hbm.at[pl.ds(r0, RCH)], sidx)
            pltpu.sync_copy(dst_hbm.at[pl.ds(r0, RCH)], didx)
            for g in range(RCH // 2):
                j0 = g * 2
                cps = [pltpu.async_copy(table.at[sidx.at[j0 + j]], rows.at[j], sem)
                       for j in range(2)]
                for cp in cps:
                    cp.wait()
                for j in range(2):
                    pltpu.sync_copy(rows.at[j], acc.at[didx.at[j0 + j]], add=True)
            return carry

        lax.fori_loop(0, rpt // RCH, body, 0)
        plsc.subcore_barrier()
        pltpu.sync_copy(acc.at[pl.ds(s * NPT, NPT)],
                        out_hbm.at[c].at[pl.ds(s * NPT, NPT)])

    return agg


_agg_es = _make_agg(True)
_agg_fs = _make_agg(False)


def _stage_a(deg_ref, x_ref, hs1_ref, norms_ref):
    ns = lax.rsqrt(jnp.clip(deg_ref[:, 0:1], 1.0, None))
    nd = lax.rsqrt(jnp.clip(deg_ref[:, 1:2], 1.0, None))
    norms_ref[:, 0:1] = ns
    norms_ref[:, 1:2] = nd
    xs = x_ref[...] * ns
    hs1_ref[0, 0:N] = xs
    hs1_ref[1, 0:N] = xs


def _stage_b(agg_ref, norms_ref, w_ref, b_ref, g_ref, be_ref, out_ref):
    ns = norms_ref[:, 0:1]
    nd = norms_ref[:, 1:2]
    a = (agg_ref[0][:N] + agg_ref[1][:N]) * nd
    z = jnp.dot(a, w_ref[...], preferred_element_type=jnp.float32) + b_ref[...]
    m = jnp.mean(z, axis=0, keepdims=True)
    d = z - m
    v = jnp.mean(d * d, axis=0, keepdims=True)
    h = jnp.maximum(d * lax.rsqrt(v + _EPS) * g_ref[...] + be_ref[...], 0.0)
    hs = h * ns
    out_ref[0, 0:N] = hs[:, :D]
    out_ref[1, 0:N] = hs[:, D:]


def _stage_c(agg_ref, norms_ref, w2_ref, b_ref, g_ref, be_ref, w3_ref, out_ref):
    ns = norms_ref[:, 0:1]
    nd = norms_ref[:, 1:2]
    a0 = agg_ref[0][:N] * nd
    a1 = agg_ref[1][:N] * nd
    z = (jnp.dot(a0, w2_ref[0:D, :], preferred_element_type=jnp.float32)
         + jnp.dot(a1, w2_ref[D:2 * D, :], preferred_element_type=jnp.float32)
         + b_ref[...])
    m = jnp.mean(z, axis=0, keepdims=True)
    d = z - m
    v = jnp.mean(d * d, axis=0, keepdims=True)
    h = jnp.maximum(d * lax.rsqrt(v + _EPS) * g_ref[...] + be_ref[...], 0.0)
    t3 = jnp.dot(h, w3_ref[...], preferred_element_type=jnp.float32) * ns
    out_ref[0, 0:N] = t3
    out_ref[1, 0:N] = t3


def _stage_d(agg_ref, norms_ref, b3_ref, out_ref):
    nd = norms_ref[:, 1:2]
    a = agg_ref[0][:N, :40] + agg_ref[1][:N, :40]
    out_ref[...] = a * nd + b3_ref[...]


def kernel(x, edge_index, W1, b1, gamma1, beta1, W2, b2, gamma2, beta2, W3, b3):
    f32 = jnp.float32
    i32 = jnp.int32
    src = edge_index[0].astype(i32)
    dst = edge_index[1].astype(i32)
    pad = jnp.full((EP - E,), N, i32)
    src2d = jnp.concatenate([src, pad]).reshape(ROWS, CH)
    dst2d = jnp.concatenate([dst, pad]).reshape(ROWS, CH)
    edge2d = jnp.stack([src2d, dst2d])
    ones8 = jnp.zeros((CH, D), f32).at[:, 0].set(1.0)
    zrow = jnp.zeros((NPT, D), f32)

    degtab = _deg_kernel(edge2d, ones8, zrow)
    degcols = jnp.stack([degtab[0, :N, 0], degtab[1, :N, 0]], axis=1)

    hs1, norms = pl.pallas_call(
        _stage_a,
        out_shape=[jax.ShapeDtypeStruct((NC, NP_, D), f32),
                   jax.ShapeDtypeStruct((N, 2), f32)],
    )(degcols, x)

    agg1 = _agg_es(hs1, src2d, dst2d, zrow)

    hs2 = pl.pallas_call(
        _stage_b,
        out_shape=jax.ShapeDtypeStruct((NC, NP_, D), f32),
    )(agg1, norms, W1, b1.reshape(1, -1), gamma1.reshape(1, -1),
      beta1.reshape(1, -1))

    agg2 = _agg_fs(hs2, src2d, dst2d, zrow)

    W3p = jnp.zeros((256, D), f32).at[:, :40].set(W3)
    t3 = pl.pallas_call(
        _stage_c,
        out_shape=jax.ShapeDtypeStruct((NC, NP_, D), f32),
    )(agg2, norms, W2, b2.reshape(1, -1), gamma2.reshape(1, -1),
      beta2.reshape(1, -1), W3p)

    agg3 = _agg_es(t3, src2d, dst2d, zrow)

    out = pl.pallas_call(
        _stage_d,
        out_shape=jax.ShapeDtypeStruct((N, 40), f32),
    )(agg3, norms, b3.reshape(1, -1))
    return out
